# baseline (speedup 1.0000x reference)
import functools

import jax
import jax.numpy as jnp
from jax import lax
from jax.experimental import pallas as pl
from jax.experimental.pallas import tpu as pltpu
from jax.experimental.pallas import tpu_sc as plsc

N = 10000
E = 320000
D = 128
H = 128

NC = 2
NS = 16
NTILES = NC * NS
EPT = E // NTILES

CH = 40
NCH = EPT // CH
SLOTS = 5
ROUNDS = NCH // SLOTS

WB_BLK = 40
NWB = N // WB_BLK
WB_PER_SUB = (NWB + NS - 1) // NS

BLK = 2000
NBLK = N // BLK

ECHUNK = 2000
NECHUNK = EPT // ECHUNK

_MESH = plsc.VectorSubcoreMesh(core_axis_name="c", subcore_axis_name="s")
_SC_LINEAR_PARAMS = pltpu.CompilerParams(use_tc_tiling_on_sc=False)
_SC_GATHER_PARAMS = pltpu.CompilerParams(use_tc_tiling_on_sc=False,
                                         needs_layout_passes=False)


@functools.partial(
    pl.kernel,
    out_type=jax.ShapeDtypeStruct((NC * N, D), jnp.float32),
    mesh=_MESH,
    scratch_types=[
        pltpu.VMEM_SHARED((N, D), jnp.float32),
        pltpu.VMEM((2 * SLOTS, CH), jnp.int32),
        pltpu.VMEM((2 * SLOTS, CH), jnp.int32),
        pltpu.VMEM((SLOTS, CH, D), jnp.float32),
        pltpu.VMEM((WB_BLK, D), jnp.float32),
        pltpu.SemaphoreType.DMA((SLOTS,)),
        pltpu.SemaphoreType.DMA((SLOTS,)),
        pltpu.SemaphoreType.DMA((SLOTS,)),
    ],
)
def _sc_segsum(table_hbm, src_hbm, dst_hbm, out_hbm,
               acc_sh, ibufs, dbufs, rbufs, zbuf, isem, gsem, ssem):
    c = lax.axis_index("c")
    s = lax.axis_index("s")
    base = (c * NS + s) * EPT

    def fire_idx(w, k):
        pltpu.async_copy(src_hbm.at[pl.ds(base + w * CH, CH)],
                         ibufs.at[k], isem.at[k % SLOTS])
        pltpu.async_copy(dst_hbm.at[pl.ds(base + w * CH, CH)],
                         dbufs.at[k], isem.at[k % SLOTS])

    def wait_idx(w, k):
        pltpu.make_async_copy(src_hbm.at[pl.ds(base + w * CH, CH)],
                              ibufs.at[k], isem.at[k % SLOTS]).wait()
        pltpu.make_async_copy(dst_hbm.at[pl.ds(base + w * CH, CH)],
                              dbufs.at[k], isem.at[k % SLOTS]).wait()

    def fire_gather(r, k):
        pltpu.async_copy(table_hbm.at[ibufs.at[k]], rbufs.at[r], gsem.at[r])

    def wait_gather(r, k):
        pltpu.make_async_copy(table_hbm.at[ibufs.at[k]], rbufs.at[r],
                              gsem.at[r]).wait()

    def fire_scatter(r, k):
        pltpu.async_copy(rbufs.at[r], acc_sh.at[dbufs.at[k]], ssem.at[r],
                         add=True)

    def wait_scatter(r, k):
        pltpu.make_async_copy(rbufs.at[r], acc_sh.at[dbufs.at[k]],
                              ssem.at[r]).wait()

    for r in range(SLOTS):
        fire_idx(r, 2 * r)

    @pl.loop(0, WB_BLK)
    def _zr(r):
        @pl.loop(0, D, step=16)
        def _zc(j):
            zbuf[r, pl.ds(j, 16)] = jnp.zeros((16,), jnp.float32)

    @pl.loop(0, WB_PER_SUB)
    def _zb(k):
        b = s * WB_PER_SUB + k

        @pl.when(b < NWB)
        def _():
            pltpu.sync_copy(zbuf, acc_sh.at[pl.ds(b * WB_BLK, WB_BLK)])

    plsc.subcore_barrier()

    def round_body(i, p, fire_next):
        for r in range(SLOTS):
            w = i * SLOTS + r
            wait_idx(w, 2 * r + p)
            fire_gather(r, 2 * r + p)
        for r in range(SLOTS):
            w = i * SLOTS + r
            if fire_next:
                fire_idx(w + SLOTS, 2 * r + (1 - p))
            wait_gather(r, 2 * r + p)
            fire_scatter(r, 2 * r + p)
        for r in range(SLOTS):
            wait_scatter(r, 2 * r + p)

    @pl.loop(0, (ROUNDS - 2) // 2)
    def _main(j):
        round_body(2 * j, 0, fire_next=True)
        round_body(2 * j + 1, 1, fire_next=True)

    round_body(ROUNDS - 2, 0, fire_next=True)
    round_body(ROUNDS - 1, 1, fire_next=False)

    plsc.subcore_barrier()

    @pl.loop(0, WB_PER_SUB)
    def _wb(k):
        b = s * WB_PER_SUB + k

        @pl.when(b < NWB)
        def _():
            pltpu.sync_copy(acc_sh.at[pl.ds(b * WB_BLK, WB_BLK)],
                            out_hbm.at[pl.ds(c * N + b * WB_BLK, WB_BLK)])


DEGW = 16


@functools.partial(
    pl.kernel,
    out_type=jax.ShapeDtypeStruct((NC * N, DEGW), jnp.float32),
    mesh=_MESH,
    compiler_params=_SC_LINEAR_PARAMS,
    scratch_types=[
        pltpu.VMEM_SHARED((N, DEGW), jnp.float32),
        pltpu.VMEM((2 * SLOTS, CH), jnp.int32),
        pltpu.VMEM((CH, DEGW), jnp.float32),
        pltpu.VMEM((WB_BLK, DEGW), jnp.float32),
        pltpu.SemaphoreType.DMA((SLOTS,)),
        pltpu.SemaphoreType.DMA((SLOTS,)),
    ],
)
def _sc_degree(dst_hbm, out_hbm, acc_sh, dbufs, ones_v, zbuf, isem, ssem):
    c = lax.axis_index("c")
    s = lax.axis_index("s")
    base = (c * NS + s) * EPT
    vone = jnp.full((16,), 1.0, jnp.float32)

    def fire_idx(w, k):
        pltpu.async_copy(dst_hbm.at[pl.ds(base + w * CH, CH)],
                         dbufs.at[k], isem.at[k % SLOTS])

    def wait_idx(w, k):
        pltpu.make_async_copy(dst_hbm.at[pl.ds(base + w * CH, CH)],
                              dbufs.at[k], isem.at[k % SLOTS]).wait()

    def fire_scatter(k):
        pltpu.async_copy(ones_v, acc_sh.at[dbufs.at[k]], ssem.at[k % SLOTS],
                         add=True)

    def wait_scatter(k):
        pltpu.make_async_copy(ones_v, acc_sh.at[dbufs.at[k]],
                              ssem.at[k % SLOTS]).wait()

    for r in range(SLOTS):
        fire_idx(r, 2 * r)

    @pl.loop(0, CH)
    def _or(r):
        ones_v[r, pl.ds(0, 16)] = vone

    @pl.loop(0, WB_BLK)
    def _zr(r):
        zbuf[r, pl.ds(0, 16)] = jnp.zeros((16,), jnp.float32)

    @pl.loop(0, WB_PER_SUB)
    def _zb(k):
        b = s * WB_PER_SUB + k

        @pl.when(b < NWB)
        def _():
            pltpu.sync_copy(zbuf, acc_sh.at[pl.ds(b * WB_BLK, WB_BLK)])

    plsc.subcore_barrier()

    def round_body(i, p, wait_prev, fire_next):
        for r in range(SLOTS):
            w = i * SLOTS + r
            wait_idx(w, 2 * r + p)
            if wait_prev:
                wait_scatter(2 * r + (1 - p))
            fire_scatter(2 * r + p)
            if fire_next:
                fire_idx(w + SLOTS, 2 * r + (1 - p))

    round_body(0, 0, wait_prev=False, fire_next=True)

    @pl.loop(0, (ROUNDS - 2) // 2)
    def _main(j):
        round_body(2 * j + 1, 1, wait_prev=True, fire_next=True)
        round_body(2 * j + 2, 0, wait_prev=True, fire_next=True)

    round_body(ROUNDS - 1, 1, wait_prev=True, fire_next=False)
    for r in range(SLOTS):
        wait_scatter(2 * r + 1)

    plsc.subcore_barrier()

    @pl.loop(0, WB_PER_SUB)
    def _wb(k):
        b = s * WB_PER_SUB + k

        @pl.when(b < NWB)
        def _():
            pltpu.sync_copy(acc_sh.at[pl.ds(b * WB_BLK, WB_BLK)],
                            out_hbm.at[pl.ds(c * N + b * WB_BLK, WB_BLK)])


@functools.partial(
    pl.kernel,
    out_type=jax.ShapeDtypeStruct((2 * E,), jnp.float32),
    mesh=_MESH,
    compiler_params=_SC_GATHER_PARAMS,
    scratch_types=[
        pltpu.VMEM((N, 4), jnp.float32),
        pltpu.VMEM((ECHUNK,), jnp.int32),
        pltpu.VMEM((ECHUNK,), jnp.int32),
        pltpu.VMEM((ECHUNK,), jnp.float32),
        pltpu.VMEM((ECHUNK,), jnp.float32),
    ],
)
def _sc_edge_logits(pq_hbm, src_hbm, dst_hbm, out_hbm,
                    pq_v, sbuf, dbuf, o0, o1):
    c = lax.axis_index("c")
    s = lax.axis_index("s")
    pltpu.sync_copy(pq_hbm, pq_v)
    base = (c * NS + s) * EPT
    col0 = jnp.full((16,), 0, jnp.int32)
    col1 = jnp.full((16,), 1, jnp.int32)
    col2 = jnp.full((16,), 2, jnp.int32)
    col3 = jnp.full((16,), 3, jnp.int32)

    @pl.loop(0, NECHUNK)
    def _win(i):
        off = base + i * ECHUNK
        pltpu.sync_copy(src_hbm.at[pl.ds(off, ECHUNK)], sbuf)
        pltpu.sync_copy(dst_hbm.at[pl.ds(off, ECHUNK)], dbuf)

        @pl.loop(0, ECHUNK // 16)
        def _vec(j):
            sv = sbuf[pl.ds(j * 16, 16)]
            dv = dbuf[pl.ds(j * 16, 16)]
            l0 = (plsc.load_gather(pq_v, [sv, col0])
                  + plsc.load_gather(pq_v, [dv, col2]))
            l1 = (plsc.load_gather(pq_v, [sv, col1])
                  + plsc.load_gather(pq_v, [dv, col3]))
            o0[pl.ds(j * 16, 16)] = l0
            o1[pl.ds(j * 16, 16)] = l1

        pltpu.sync_copy(o0, out_hbm.at[pl.ds(off, ECHUNK)])
        pltpu.sync_copy(o1, out_hbm.at[pl.ds(E + off, ECHUNK)])


def _layer_body(p0_ref, p1_ref, deg_ref, x_ref, wl_ref, b_ref, wr_ref, out_ref):
    summed = p0_ref[...] + p1_ref[...]
    deg = jnp.maximum(deg_ref[...], 1.0)
    mean = summed / deg
    h = (lax.dot_general(mean, wl_ref[...], (((1,), (1,)), ((), ())),
                         preferred_element_type=jnp.float32,
                         precision=lax.Precision.HIGHEST)
         + b_ref[...]
         + lax.dot_general(x_ref[...], wr_ref[...], (((1,), (1,)), ((), ())),
                           preferred_element_type=jnp.float32,
                           precision=lax.Precision.HIGHEST))
    out_ref[...] = jnp.maximum(h, 0.0)


def _tc_layer(acc, deg, x, Wl, b, Wr):
    return pl.pallas_call(
        _layer_body,
        grid=(NBLK,),
        in_specs=[
            pl.BlockSpec((BLK, D), lambda i: (i, 0)),
            pl.BlockSpec((BLK, D), lambda i: (i + NBLK, 0)),
            pl.BlockSpec((BLK, 1), lambda i: (i, 0)),
            pl.BlockSpec((BLK, D), lambda i: (i, 0)),
            pl.BlockSpec((H, D), lambda i: (0, 0)),
            pl.BlockSpec((1, H), lambda i: (0, 0)),
            pl.BlockSpec((H, D), lambda i: (0, 0)),
        ],
        out_specs=pl.BlockSpec((BLK, H), lambda i: (i, 0)),
        out_shape=jax.ShapeDtypeStruct((N, H), jnp.float32),
    )(acc, acc, deg, x, Wl, b.reshape(1, H), Wr)


def _pq_body(p0_ref, p1_ref, deg_ref, h_ref, wl_ref, b_ref, wr_ref,
             w4_ref, bc4_ref, out_ref):
    summed = p0_ref[...] + p1_ref[...]
    deg = jnp.maximum(deg_ref[...], 1.0)
    mean = summed / deg
    h = (lax.dot_general(mean, wl_ref[...], (((1,), (1,)), ((), ())),
                         preferred_element_type=jnp.float32,
                         precision=lax.Precision.HIGHEST)
         + b_ref[...]
         + lax.dot_general(h_ref[...], wr_ref[...], (((1,), (1,)), ((), ())),
                           preferred_element_type=jnp.float32,
                           precision=lax.Precision.HIGHEST))
    h = jnp.maximum(h, 0.0)
    out_ref[...] = lax.dot_general(h, w4_ref[...], (((1,), (1,)), ((), ())),
                                   preferred_element_type=jnp.float32,
                                   precision=lax.Precision.HIGHEST) + bc4_ref[...]


def _tc_pq(acc2, deg, h1, Wl, b, Wr, W4, bc4):
    return pl.pallas_call(
        _pq_body,
        grid=(NBLK,),
        in_specs=[
            pl.BlockSpec((BLK, D), lambda i: (i, 0)),
            pl.BlockSpec((BLK, D), lambda i: (i + NBLK, 0)),
            pl.BlockSpec((BLK, 1), lambda i: (i, 0)),
            pl.BlockSpec((BLK, H), lambda i: (i, 0)),
            pl.BlockSpec((H, H), lambda i: (0, 0)),
            pl.BlockSpec((1, H), lambda i: (0, 0)),
            pl.BlockSpec((H, H), lambda i: (0, 0)),
            pl.BlockSpec((4, H), lambda i: (0, 0)),
            pl.BlockSpec((1, 4), lambda i: (0, 0)),
        ],
        out_specs=pl.BlockSpec((BLK, 4), lambda i: (i, 0)),
        out_shape=jax.ShapeDtypeStruct((N, 4), jnp.float32),
    )(acc2, acc2, deg, h1, Wl, b.reshape(1, H), Wr, W4, bc4.reshape(1, 4))


def kernel(x, edge_index, W1l, b1, W1r, W2l, b2, W2r, Wc, bc):
    src = edge_index[0]
    dst = edge_index[1]

    deg2 = _sc_degree(dst)
    deg = deg2[:N, :1] + deg2[N:, :1]

    acc1 = _sc_segsum(x, src, dst)
    h1 = _tc_layer(acc1, deg, x, W1l, b1, W1r)

    acc2 = _sc_segsum(h1, src, dst)
    W4 = jnp.concatenate([Wc[:, :H], Wc[:, H:]], axis=0)
    bc4 = jnp.concatenate([bc, jnp.zeros((2,), jnp.float32)])
    pq = _tc_pq(acc2, deg, h1, W2l, b2, W2r, W4, bc4)

    flat = _sc_edge_logits(pq, src, dst)
    return flat.reshape(2, E).T


# speedup vs baseline: 1.1120x; 1.1120x over previous
import functools

import jax
import jax.numpy as jnp
from jax import lax
from jax.experimental import pallas as pl
from jax.experimental.pallas import tpu as pltpu
from jax.experimental.pallas import tpu_sc as plsc

N = 10000
E = 320000
D = 128
H = 128

NC = 2
NS = 16
NTILES = NC * NS
EPT = E // NTILES

CH = 40
NCH = EPT // CH
SLOTS = 5
ROUNDS = NCH // SLOTS

WB_BLK = 40
NWB = N // WB_BLK
WB_PER_SUB = (NWB + NS - 1) // NS

BLK = 2000
NBLK = N // BLK

ECHUNK = 2000
NECHUNK = EPT // ECHUNK

_MESH = plsc.VectorSubcoreMesh(core_axis_name="c", subcore_axis_name="s")
_SC_LINEAR_PARAMS = pltpu.CompilerParams(use_tc_tiling_on_sc=False)
_SC_GATHER_PARAMS = pltpu.CompilerParams(use_tc_tiling_on_sc=False,
                                         needs_layout_passes=False)


@functools.partial(
    pl.kernel,
    out_type=jax.ShapeDtypeStruct((NC * N, D), jnp.float32),
    mesh=_MESH,
    scratch_types=[
        pltpu.VMEM_SHARED((N, D), jnp.float32),
        pltpu.VMEM((2 * SLOTS, CH), jnp.int32),
        pltpu.VMEM((2 * SLOTS, CH), jnp.int32),
        pltpu.VMEM((2 * SLOTS - 1, CH, D), jnp.float32),
        pltpu.SemaphoreType.DMA((SLOTS,)),
        pltpu.SemaphoreType.DMA((SLOTS,)),
        pltpu.SemaphoreType.DMA((SLOTS,)),
    ],
)
def _sc_segsum(table_hbm, src_hbm, dst_hbm, out_hbm,
               acc_sh, ibufs, dbufs, rbufs, isem, gsem, ssem):
    c = lax.axis_index("c")
    s = lax.axis_index("s")
    base = (c * NS + s) * EPT
    LAST = SLOTS - 1

    def kr(r, p):
        return 2 * r + p if r < LAST else 2 * LAST

    def fire_idx(w, k):
        pltpu.async_copy(src_hbm.at[pl.ds(base + w * CH, CH)],
                         ibufs.at[k], isem.at[k % SLOTS])
        pltpu.async_copy(dst_hbm.at[pl.ds(base + w * CH, CH)],
                         dbufs.at[k], isem.at[k % SLOTS])

    def wait_idx(w, k):
        pltpu.make_async_copy(src_hbm.at[pl.ds(base + w * CH, CH)],
                              ibufs.at[k], isem.at[k % SLOTS]).wait()
        pltpu.make_async_copy(dst_hbm.at[pl.ds(base + w * CH, CH)],
                              dbufs.at[k], isem.at[k % SLOTS]).wait()

    def fire_gather(r, p):
        pltpu.async_copy(table_hbm.at[ibufs.at[2 * r + p]], rbufs.at[kr(r, p)],
                         gsem.at[r])

    def wait_gather(r, p):
        pltpu.make_async_copy(table_hbm.at[ibufs.at[2 * r + p]],
                              rbufs.at[kr(r, p)], gsem.at[r]).wait()

    def fire_scatter(r, p):
        pltpu.async_copy(rbufs.at[kr(r, p)], acc_sh.at[dbufs.at[2 * r + p]],
                         ssem.at[r], add=True)

    def wait_scatter(r, p):
        pltpu.make_async_copy(rbufs.at[kr(r, p)], acc_sh.at[dbufs.at[2 * r + p]],
                              ssem.at[r]).wait()

    for r in range(SLOTS):
        fire_idx(r, 2 * r)

    @pl.loop(0, WB_BLK)
    def _zr(r):
        @pl.loop(0, D, step=16)
        def _zc(j):
            rbufs[0, r, pl.ds(j, 16)] = jnp.zeros((16,), jnp.float32)

    @pl.loop(0, WB_PER_SUB)
    def _zb(k):
        b = s * WB_PER_SUB + k

        @pl.when(b < NWB)
        def _():
            pltpu.sync_copy(rbufs.at[0], acc_sh.at[pl.ds(b * WB_BLK, WB_BLK)])

    plsc.subcore_barrier()

    def round_body(i, p, wait_prev, fire_next):
        for r in range(SLOTS):
            w = i * SLOTS + r
            if r == LAST and wait_prev:
                wait_scatter(r, 1 - p)
            wait_idx(w, 2 * r + p)
            fire_gather(r, p)
        for r in range(SLOTS):
            w = i * SLOTS + r
            if r < LAST and wait_prev:
                wait_scatter(r, 1 - p)
            if fire_next:
                fire_idx(w + SLOTS, 2 * r + (1 - p))
        for r in range(SLOTS):
            wait_gather(r, p)
            fire_scatter(r, p)

    round_body(0, 0, wait_prev=False, fire_next=True)

    @pl.loop(0, (ROUNDS - 2) // 2)
    def _main(j):
        round_body(2 * j + 1, 1, wait_prev=True, fire_next=True)
        round_body(2 * j + 2, 0, wait_prev=True, fire_next=True)

    round_body(ROUNDS - 1, 1, wait_prev=True, fire_next=False)
    for r in range(SLOTS):
        wait_scatter(r, 1)

    plsc.subcore_barrier()

    @pl.loop(0, WB_PER_SUB)
    def _wb(k):
        b = s * WB_PER_SUB + k

        @pl.when(b < NWB)
        def _():
            pltpu.sync_copy(acc_sh.at[pl.ds(b * WB_BLK, WB_BLK)],
                            out_hbm.at[pl.ds(c * N + b * WB_BLK, WB_BLK)])


DEGW = 16


@functools.partial(
    pl.kernel,
    out_type=jax.ShapeDtypeStruct((NC * N, DEGW), jnp.float32),
    mesh=_MESH,
    compiler_params=_SC_LINEAR_PARAMS,
    scratch_types=[
        pltpu.VMEM_SHARED((N, DEGW), jnp.float32),
        pltpu.VMEM((2 * SLOTS, CH), jnp.int32),
        pltpu.VMEM((CH, DEGW), jnp.float32),
        pltpu.VMEM((WB_BLK, DEGW), jnp.float32),
        pltpu.SemaphoreType.DMA((SLOTS,)),
        pltpu.SemaphoreType.DMA((SLOTS,)),
    ],
)
def _sc_degree(dst_hbm, out_hbm, acc_sh, dbufs, ones_v, zbuf, isem, ssem):
    c = lax.axis_index("c")
    s = lax.axis_index("s")
    base = (c * NS + s) * EPT
    vone = jnp.full((16,), 1.0, jnp.float32)

    def fire_idx(w, k):
        pltpu.async_copy(dst_hbm.at[pl.ds(base + w * CH, CH)],
                         dbufs.at[k], isem.at[k % SLOTS])

    def wait_idx(w, k):
        pltpu.make_async_copy(dst_hbm.at[pl.ds(base + w * CH, CH)],
                              dbufs.at[k], isem.at[k % SLOTS]).wait()

    def fire_scatter(k):
        pltpu.async_copy(ones_v, acc_sh.at[dbufs.at[k]], ssem.at[k % SLOTS],
                         add=True)

    def wait_scatter(k):
        pltpu.make_async_copy(ones_v, acc_sh.at[dbufs.at[k]],
                              ssem.at[k % SLOTS]).wait()

    for r in range(SLOTS):
        fire_idx(r, 2 * r)

    @pl.loop(0, CH)
    def _or(r):
        ones_v[r, pl.ds(0, 16)] = vone

    @pl.loop(0, WB_BLK)
    def _zr(r):
        zbuf[r, pl.ds(0, 16)] = jnp.zeros((16,), jnp.float32)

    @pl.loop(0, WB_PER_SUB)
    def _zb(k):
        b = s * WB_PER_SUB + k

        @pl.when(b < NWB)
        def _():
            pltpu.sync_copy(zbuf, acc_sh.at[pl.ds(b * WB_BLK, WB_BLK)])

    plsc.subcore_barrier()

    def round_body(i, p, wait_prev, fire_next):
        for r in range(SLOTS):
            w = i * SLOTS + r
            wait_idx(w, 2 * r + p)
            if wait_prev:
                wait_scatter(2 * r + (1 - p))
            fire_scatter(2 * r + p)
            if fire_next:
                fire_idx(w + SLOTS, 2 * r + (1 - p))

    round_body(0, 0, wait_prev=False, fire_next=True)

    @pl.loop(0, (ROUNDS - 2) // 2)
    def _main(j):
        round_body(2 * j + 1, 1, wait_prev=True, fire_next=True)
        round_body(2 * j + 2, 0, wait_prev=True, fire_next=True)

    round_body(ROUNDS - 1, 1, wait_prev=True, fire_next=False)
    for r in range(SLOTS):
        wait_scatter(2 * r + 1)

    plsc.subcore_barrier()

    @pl.loop(0, WB_PER_SUB)
    def _wb(k):
        b = s * WB_PER_SUB + k

        @pl.when(b < NWB)
        def _():
            pltpu.sync_copy(acc_sh.at[pl.ds(b * WB_BLK, WB_BLK)],
                            out_hbm.at[pl.ds(c * N + b * WB_BLK, WB_BLK)])


@functools.partial(
    pl.kernel,
    out_type=jax.ShapeDtypeStruct((2 * E,), jnp.float32),
    mesh=_MESH,
    compiler_params=_SC_GATHER_PARAMS,
    scratch_types=[
        pltpu.VMEM((N, 4), jnp.float32),
        pltpu.VMEM((ECHUNK,), jnp.int32),
        pltpu.VMEM((ECHUNK,), jnp.int32),
        pltpu.VMEM((ECHUNK,), jnp.float32),
        pltpu.VMEM((ECHUNK,), jnp.float32),
    ],
)
def _sc_edge_logits(pq_hbm, src_hbm, dst_hbm, out_hbm,
                    pq_v, sbuf, dbuf, o0, o1):
    c = lax.axis_index("c")
    s = lax.axis_index("s")
    pltpu.sync_copy(pq_hbm, pq_v)
    base = (c * NS + s) * EPT
    col0 = jnp.full((16,), 0, jnp.int32)
    col1 = jnp.full((16,), 1, jnp.int32)
    col2 = jnp.full((16,), 2, jnp.int32)
    col3 = jnp.full((16,), 3, jnp.int32)

    @pl.loop(0, NECHUNK)
    def _win(i):
        off = base + i * ECHUNK
        pltpu.sync_copy(src_hbm.at[pl.ds(off, ECHUNK)], sbuf)
        pltpu.sync_copy(dst_hbm.at[pl.ds(off, ECHUNK)], dbuf)

        @pl.loop(0, ECHUNK // 16)
        def _vec(j):
            sv = sbuf[pl.ds(j * 16, 16)]
            dv = dbuf[pl.ds(j * 16, 16)]
            l0 = (plsc.load_gather(pq_v, [sv, col0])
                  + plsc.load_gather(pq_v, [dv, col2]))
            l1 = (plsc.load_gather(pq_v, [sv, col1])
                  + plsc.load_gather(pq_v, [dv, col3]))
            o0[pl.ds(j * 16, 16)] = l0
            o1[pl.ds(j * 16, 16)] = l1

        pltpu.sync_copy(o0, out_hbm.at[pl.ds(off, ECHUNK)])
        pltpu.sync_copy(o1, out_hbm.at[pl.ds(E + off, ECHUNK)])


def _layer_body(p0_ref, p1_ref, deg_ref, x_ref, wl_ref, b_ref, wr_ref, out_ref):
    summed = p0_ref[...] + p1_ref[...]
    deg = jnp.maximum(deg_ref[...], 1.0)
    mean = summed / deg
    h = (lax.dot_general(mean, wl_ref[...], (((1,), (1,)), ((), ())),
                         preferred_element_type=jnp.float32,
                         precision=lax.Precision.HIGHEST)
         + b_ref[...]
         + lax.dot_general(x_ref[...], wr_ref[...], (((1,), (1,)), ((), ())),
                           preferred_element_type=jnp.float32,
                           precision=lax.Precision.HIGHEST))
    out_ref[...] = jnp.maximum(h, 0.0)


def _tc_layer(acc, deg, x, Wl, b, Wr):
    return pl.pallas_call(
        _layer_body,
        grid=(NBLK,),
        in_specs=[
            pl.BlockSpec((BLK, D), lambda i: (i, 0)),
            pl.BlockSpec((BLK, D), lambda i: (i + NBLK, 0)),
            pl.BlockSpec((BLK, 1), lambda i: (i, 0)),
            pl.BlockSpec((BLK, D), lambda i: (i, 0)),
            pl.BlockSpec((H, D), lambda i: (0, 0)),
            pl.BlockSpec((1, H), lambda i: (0, 0)),
            pl.BlockSpec((H, D), lambda i: (0, 0)),
        ],
        out_specs=pl.BlockSpec((BLK, H), lambda i: (i, 0)),
        out_shape=jax.ShapeDtypeStruct((N, H), jnp.float32),
    )(acc, acc, deg, x, Wl, b.reshape(1, H), Wr)


def _pq_body(p0_ref, p1_ref, deg_ref, h_ref, wl_ref, b_ref, wr_ref,
             w4_ref, bc4_ref, out_ref):
    summed = p0_ref[...] + p1_ref[...]
    deg = jnp.maximum(deg_ref[...], 1.0)
    mean = summed / deg
    h = (lax.dot_general(mean, wl_ref[...], (((1,), (1,)), ((), ())),
                         preferred_element_type=jnp.float32,
                         precision=lax.Precision.HIGHEST)
         + b_ref[...]
         + lax.dot_general(h_ref[...], wr_ref[...], (((1,), (1,)), ((), ())),
                           preferred_element_type=jnp.float32,
                           precision=lax.Precision.HIGHEST))
    h = jnp.maximum(h, 0.0)
    out_ref[...] = lax.dot_general(h, w4_ref[...], (((1,), (1,)), ((), ())),
                                   preferred_element_type=jnp.float32,
                                   precision=lax.Precision.HIGHEST) + bc4_ref[...]


def _tc_pq(acc2, deg, h1, Wl, b, Wr, W4, bc4):
    return pl.pallas_call(
        _pq_body,
        grid=(NBLK,),
        in_specs=[
            pl.BlockSpec((BLK, D), lambda i: (i, 0)),
            pl.BlockSpec((BLK, D), lambda i: (i + NBLK, 0)),
            pl.BlockSpec((BLK, 1), lambda i: (i, 0)),
            pl.BlockSpec((BLK, H), lambda i: (i, 0)),
            pl.BlockSpec((H, H), lambda i: (0, 0)),
            pl.BlockSpec((1, H), lambda i: (0, 0)),
            pl.BlockSpec((H, H), lambda i: (0, 0)),
            pl.BlockSpec((4, H), lambda i: (0, 0)),
            pl.BlockSpec((1, 4), lambda i: (0, 0)),
        ],
        out_specs=pl.BlockSpec((BLK, 4), lambda i: (i, 0)),
        out_shape=jax.ShapeDtypeStruct((N, 4), jnp.float32),
    )(acc2, acc2, deg, h1, Wl, b.reshape(1, H), Wr, W4, bc4.reshape(1, 4))


def kernel(x, edge_index, W1l, b1, W1r, W2l, b2, W2r, Wc, bc):
    src = edge_index[0]
    dst = edge_index[1]

    deg2 = _sc_degree(dst)
    deg = deg2[:N, :1] + deg2[N:, :1]

    acc1 = _sc_segsum(x, src, dst)
    h1 = _tc_layer(acc1, deg, x, W1l, b1, W1r)

    acc2 = _sc_segsum(h1, src, dst)
    W4 = jnp.concatenate([Wc[:, :H], Wc[:, H:]], axis=0)
    bc4 = jnp.concatenate([bc, jnp.zeros((2,), jnp.float32)])
    pq = _tc_pq(acc2, deg, h1, W2l, b2, W2r, W4, bc4)

    flat = _sc_edge_logits(pq, src, dst)
    return flat.reshape(2, E).T
